# baseline (speedup 1.0000x reference)
import functools

import jax
import jax.numpy as jnp
from jax import lax
from jax.experimental import pallas as pl
from jax.experimental.pallas import tpu as pltpu
from jax.experimental.pallas import tpu_sc as plsc

_N = 10000
_D = 128
_E = 320000
_R = 3
_K = 64
_NC = 2
_NS = 16
_NW = _NC * _NS
_EPT = _E // _NW
_FC = _EPT // _K
_REM = _EPT - _FC * _K
_NB = 4
_NG = _FC // _NB
_ZR = 400
_CQ = _N // _ZR


def _sc_aggregate(x, e0, e1, e2, zeros):
    mesh = plsc.VectorSubcoreMesh(core_axis_name="c", subcore_axis_name="s")

    @functools.partial(
        pl.kernel,
        mesh=mesh,
        out_type=(
            jax.ShapeDtypeStruct((_R, _NC, _N, _D), jnp.float32),
            jax.ShapeDtypeStruct((_R * _CQ * _NW * _ZR,), jnp.float32),
        ),
        scratch_types=[
            [pltpu.VMEM((_K, _D), jnp.float32) for _ in range(_NB)],
            [pltpu.VMEM((_K,), jnp.int32) for _ in range(_NB)],
            [pltpu.VMEM((_K,), jnp.int32) for _ in range(_NB)],
            [pltpu.VMEM((_K,), jnp.int32) for _ in range(_NB)],
            pltpu.VMEM((_REM,), jnp.int32),
            pltpu.VMEM((_REM,), jnp.int32),
            pltpu.VMEM((_N,), jnp.float32),
            pltpu.VMEM_SHARED((_N, _D), jnp.float32),
            [pltpu.SemaphoreType.DMA for _ in range(_NB)],
            [pltpu.SemaphoreType.DMA for _ in range(_NB)],
            [pltpu.SemaphoreType.DMA for _ in range(_NB)],
        ],
        compiler_params=pltpu.CompilerParams(needs_layout_passes=False),
    )
    def body(x_hbm, e0_hbm, e1_hbm, e2_hbm, zeros_hbm, agg_out, deg_out,
             rows, srcb, dstraw, dstb, src_rem, dst_rem, deg_v,
             agg_sh, sem_g, sem_s, sem_i):
        c = lax.axis_index("c")
        s = lax.axis_index("s")
        wid = s * _NC + c
        z16 = jnp.zeros((16,), jnp.float32)
        one16 = jnp.ones((16,), jnp.float32)

        for r, e_hbm in enumerate((e0_hbm, e1_hbm, e2_hbm)):
            sbase = pl.multiple_of(wid * _EPT, 8)
            dbase = pl.multiple_of(_E + wid * _EPT, 8)

            def _idx_start(k, b):
                so = pl.multiple_of(sbase + k * _K, 8)
                do = pl.multiple_of(dbase + k * _K, 8)
                pltpu.async_copy(e_hbm.at[pl.ds(so, _K)], srcb[b], sem_i[b])
                pltpu.async_copy(e_hbm.at[pl.ds(do, _K)], dstraw[b], sem_i[b])

            def _idx_wait(k, b):
                so = pl.multiple_of(sbase + k * _K, 8)
                do = pl.multiple_of(dbase + k * _K, 8)
                pltpu.make_async_copy(e_hbm.at[pl.ds(so, _K)], srcb[b],
                                      sem_i[b]).wait()
                pltpu.make_async_copy(e_hbm.at[pl.ds(do, _K)], dstraw[b],
                                      sem_i[b]).wait()

            def _gather(b):
                pltpu.async_copy(x_hbm.at[srcb[b]], rows[b], sem_g[b])

            def _gather_wait(b):
                pltpu.make_async_copy(x_hbm.at[srcb[b]], rows[b],
                                      sem_g[b]).wait()

            def _scatter_wait(b):
                pltpu.make_async_copy(rows[b], agg_sh.at[dstb[b]],
                                      sem_s[b]).wait()

            def _stage(b):
                for j in range(_K // 16):
                    dj = dstraw[b][pl.ds(j * 16, 16)]
                    plsc.addupdate_scatter(deg_v, [dj], one16)
                    dstb[b][pl.ds(j * 16, 16)] = dj

            for b in range(_NB):
                _idx_start(b, b)
            _idx_wait(0, 0)
            _gather(0)
            _idx_wait(1, 1)
            _gather(1)

            for cq in range(2):
                q = cq * _NS + s

                @pl.when(q < _CQ)
                def _():
                    row = pl.multiple_of(q * _ZR, _ZR)
                    pltpu.sync_copy(zeros_hbm, agg_sh.at[pl.ds(row, _ZR)])

            def _zero_deg(i, carry):
                deg_v[pl.ds(i * 16, 16)] = z16
                return carry

            lax.fori_loop(0, _N // 16, _zero_deg, 0)
            plsc.subcore_barrier()

            def _phase(k, b, g):
                b2 = (b + 2) % _NB
                if b >= 2:
                    _scatter_wait(b2)
                else:

                    @pl.when(g > 0)
                    def _():
                        _scatter_wait(b2)

                if b >= 2:

                    @pl.when(g + 1 < _NG)
                    def _():
                        _idx_wait(k + 2, b2)
                        _gather(b2)
                else:
                    _idx_wait(k + 2, b2)
                    _gather(b2)
                _gather_wait(b)
                _stage(b)
                pltpu.async_copy(rows[b], agg_sh.at[dstb[b]], sem_s[b],
                                 add=True)

                @pl.when(g + 1 < _NG)
                def _():
                    _idx_start(k + 4, b)

            def _group(g, carry):
                for b in range(_NB):
                    _phase(4 * g + b, b, g)
                return carry

            lax.fori_loop(0, _NG, _group, 0)

            roffs = pl.multiple_of(sbase + _FC * _K, 8)
            roffd = pl.multiple_of(dbase + _FC * _K, 8)
            pltpu.sync_copy(e_hbm.at[pl.ds(roffs, _REM)], src_rem)
            pltpu.sync_copy(e_hbm.at[pl.ds(roffd, _REM)], dst_rem)
            rows_rem = rows[0].at[pl.ds(0, _REM)]
            pltpu.async_copy(x_hbm.at[src_rem], rows_rem, sem_g[0]).wait()
            dj = dst_rem[...]
            plsc.addupdate_scatter(deg_v, [dj], one16)
            pltpu.async_copy(rows_rem, agg_sh.at[dst_rem], sem_s[0], add=True)
            _scatter_wait(2)
            _scatter_wait(3)
            pltpu.make_async_copy(rows_rem, agg_sh.at[dst_rem],
                                  sem_s[0]).wait()
            plsc.subcore_barrier()

            for cq in range(2):
                q = cq * _NS + s

                @pl.when(q < _CQ)
                def _():
                    row = pl.multiple_of(q * _ZR, _ZR)
                    pltpu.sync_copy(agg_sh.at[pl.ds(row, _ZR)],
                                    agg_out.at[r, c, pl.ds(row, _ZR)])

            for q in range(_CQ):
                off = pl.multiple_of(((r * _CQ + q) * _NW + wid) * _ZR, 8)
                pltpu.sync_copy(deg_v.at[pl.ds(q * _ZR, _ZR)],
                                deg_out.at[pl.ds(off, _ZR)])

    return body(x, e0, e1, e2, zeros)


_BN = 400
_GB = _N // _BN


def _tc_dense(x, agg, deg, w_self, w_neigh, bias):
    def body(x_ref, agg_ref, deg_ref, ws_ref, wn_ref, b_ref, out_ref):
        xb = x_ref[...]
        ws = ws_ref[...]
        wn = wn_ref[...]
        b = b_ref[...]
        acc = jnp.dot(xb, ws[0] + ws[1] + ws[2],
                      preferred_element_type=jnp.float32)
        degs = jnp.sum(deg_ref[...], axis=(1, 2))
        inv = 1.0 / jnp.maximum(degs, 1.0)
        ag = agg_ref[...]
        for r in range(_R):
            h = (ag[r, 0] + ag[r, 1]) * inv[r][:, None]
            acc = acc + jnp.dot(h, wn[r], preferred_element_type=jnp.float32)
        acc = acc + (b[0] + b[1] + b[2])[None, :]
        out_ref[...] = acc

    return pl.pallas_call(
        body,
        grid=(_GB,),
        in_specs=[
            pl.BlockSpec((_BN, _D), lambda i: (i, 0)),
            pl.BlockSpec((_R, _NC, _BN, _D), lambda i: (0, 0, i, 0)),
            pl.BlockSpec((_R, 1, _NW, _ZR), lambda i: (0, i, 0, 0)),
            pl.BlockSpec((_R, _D, _D), lambda i: (0, 0, 0)),
            pl.BlockSpec((_R, _D, _D), lambda i: (0, 0, 0)),
            pl.BlockSpec((_R, _D), lambda i: (0, 0)),
        ],
        out_specs=pl.BlockSpec((_BN, _D), lambda i: (i, 0)),
        out_shape=jax.ShapeDtypeStruct((_N, _D), jnp.float32),
    )(x, agg, deg, w_self, w_neigh, bias)


def kernel(x, edge_index_r0, edge_index_r1, edge_index_r2,
           W_self_r0, W_neigh_r0, b_r0,
           W_self_r1, W_neigh_r1, b_r1,
           W_self_r2, W_neigh_r2, b_r2):
    e0 = edge_index_r0.reshape(-1)
    e1 = edge_index_r1.reshape(-1)
    e2 = edge_index_r2.reshape(-1)
    zeros = jnp.zeros((_ZR, _D), jnp.float32)
    agg, deg_flat = _sc_aggregate(x, e0, e1, e2, zeros)
    deg = deg_flat.reshape(_R, _CQ, _NW, _ZR)
    w_self = jnp.stack([W_self_r0, W_self_r1, W_self_r2])
    w_neigh = jnp.stack([W_neigh_r0, W_neigh_r1, W_neigh_r2])
    bias = jnp.stack([b_r0, b_r1, b_r2])
    return _tc_dense(x, agg, deg, w_self, w_neigh, bias)


# speedup vs baseline: 1.0517x; 1.0517x over previous
import functools

import jax
import jax.numpy as jnp
from jax import lax
from jax.experimental import pallas as pl
from jax.experimental.pallas import tpu as pltpu
from jax.experimental.pallas import tpu_sc as plsc

_N = 10000
_D = 128
_E = 320000
_R = 3
_K = 96
_NC = 2
_NS = 16
_NW = _NC * _NS
_EPT = _E // _NW
_FC = _EPT // _K
_REM = _EPT - _FC * _K
_NB = 3
_NG = 34
_ZR = 400
_CQ = _N // _ZR


def _sc_aggregate(x, e0, e1, e2, zeros):
    mesh = plsc.VectorSubcoreMesh(core_axis_name="c", subcore_axis_name="s")

    @functools.partial(
        pl.kernel,
        mesh=mesh,
        out_type=(
            jax.ShapeDtypeStruct((_R, _NC, _N, _D), jnp.float32),
            jax.ShapeDtypeStruct((_R * _CQ * _NW * _ZR,), jnp.float32),
        ),
        scratch_types=[
            [pltpu.VMEM((_K, _D), jnp.float32) for _ in range(_NB)],
            [pltpu.VMEM((_K,), jnp.int32) for _ in range(_NB)],
            [pltpu.VMEM((_K,), jnp.int32) for _ in range(_NB)],
            [pltpu.VMEM((_K,), jnp.int32) for _ in range(_NB)],
            pltpu.VMEM((_REM,), jnp.int32),
            pltpu.VMEM((_REM,), jnp.int32),
            pltpu.VMEM((_N,), jnp.float32),
            pltpu.VMEM_SHARED((_N, _D), jnp.float32),
            [pltpu.SemaphoreType.DMA for _ in range(_NB)],
            [pltpu.SemaphoreType.DMA for _ in range(_NB)],
            [pltpu.SemaphoreType.DMA for _ in range(_NB)],
            pltpu.SemaphoreType.DMA,
        ],
        compiler_params=pltpu.CompilerParams(needs_layout_passes=False),
    )
    def body(x_hbm, e0_hbm, e1_hbm, e2_hbm, zeros_hbm, agg_out, deg_out,
             rows, srcb, dstraw, dstb, src_rem, dst_rem, deg_v,
             agg_sh, sem_g, sem_s, sem_i, sem_z):
        c = lax.axis_index("c")
        s = lax.axis_index("s")
        wid = s * _NC + c
        z16 = jnp.zeros((16,), jnp.float32)
        one16 = jnp.ones((16,), jnp.float32)

        for r, e_hbm in enumerate((e0_hbm, e1_hbm, e2_hbm)):
            sbase = pl.multiple_of(wid * _EPT, 8)
            dbase = pl.multiple_of(_E + wid * _EPT, 8)

            def _idx_start(k, b):
                so = pl.multiple_of(sbase + k * _K, 8)
                do = pl.multiple_of(dbase + k * _K, 8)
                pltpu.async_copy(e_hbm.at[pl.ds(so, _K)], srcb[b], sem_i[b])
                pltpu.async_copy(e_hbm.at[pl.ds(do, _K)], dstraw[b], sem_i[b])

            def _idx_wait(k, b):
                so = pl.multiple_of(sbase + k * _K, 8)
                do = pl.multiple_of(dbase + k * _K, 8)
                pltpu.make_async_copy(e_hbm.at[pl.ds(so, _K)], srcb[b],
                                      sem_i[b]).wait()
                pltpu.make_async_copy(e_hbm.at[pl.ds(do, _K)], dstraw[b],
                                      sem_i[b]).wait()

            def _gather(b):
                pltpu.async_copy(x_hbm.at[srcb[b]], rows[b], sem_g[b])

            def _gather_wait(b):
                pltpu.make_async_copy(x_hbm.at[srcb[b]], rows[b],
                                      sem_g[b]).wait()

            def _scatter_wait(b):
                pltpu.make_async_copy(rows[b], agg_sh.at[dstb[b]],
                                      sem_s[b]).wait()

            def _stage(b):
                for j in range(_K // 16):
                    dj = dstraw[b][pl.ds(j * 16, 16)]
                    plsc.addupdate_scatter(deg_v, [dj], one16)
                    dstb[b][pl.ds(j * 16, 16)] = dj

            for b in range(_NB):
                _idx_start(b, b)
            _idx_wait(0, 0)
            _gather(0)

            for cq in range(2):
                q = cq * _NS + s

                @pl.when(q < _CQ)
                def _():
                    row = pl.multiple_of(q * _ZR, _ZR)
                    pltpu.async_copy(zeros_hbm, agg_sh.at[pl.ds(row, _ZR)],
                                     sem_z)

            def _zero_deg(i, carry):
                deg_v[pl.ds(i * 16, 16)] = z16
                return carry

            lax.fori_loop(0, _N // 16, _zero_deg, 0)
            for cq in range(2):
                q = cq * _NS + s

                @pl.when(q < _CQ)
                def _():
                    row = pl.multiple_of(q * _ZR, _ZR)
                    pltpu.make_async_copy(zeros_hbm,
                                          agg_sh.at[pl.ds(row, _ZR)],
                                          sem_z).wait()
            plsc.subcore_barrier()

            def _phase(k, b, g):
                nb = (b + 1) % _NB
                if b == 2:
                    _scatter_wait(nb)
                else:

                    @pl.when(g > 0)
                    def _():
                        _scatter_wait(nb)

                _idx_wait(k + 1, nb)
                _gather(nb)
                _gather_wait(b)
                _stage(b)
                pltpu.async_copy(rows[b], agg_sh.at[dstb[b]], sem_s[b],
                                 add=True)
                if b == 2:

                    @pl.when(g + 1 < _NG)
                    def _():
                        _idx_start(k + 3, b)
                else:
                    _idx_start(k + 3, b)

            def _group(g, carry):
                for b in range(_NB):
                    _phase(3 * g + b, b, g)
                return carry

            lax.fori_loop(0, _NG, _group, 0)

            ke = _NG * _NB
            _scatter_wait(1)
            _idx_wait(ke + 1, 1)
            _gather(1)
            _gather_wait(0)
            _stage(0)
            pltpu.async_copy(rows[0], agg_sh.at[dstb[0]], sem_s[0], add=True)
            _scatter_wait(2)
            _gather_wait(1)
            _stage(1)
            pltpu.async_copy(rows[1], agg_sh.at[dstb[1]], sem_s[1], add=True)
            _scatter_wait(0)
            roffs = pl.multiple_of(sbase + _FC * _K, 8)
            roffd = pl.multiple_of(dbase + _FC * _K, 8)
            pltpu.sync_copy(e_hbm.at[pl.ds(roffs, _REM)], src_rem)
            pltpu.sync_copy(e_hbm.at[pl.ds(roffd, _REM)], dst_rem)
            rows_rem = rows[2].at[pl.ds(0, _REM)]
            pltpu.async_copy(x_hbm.at[src_rem], rows_rem, sem_g[2]).wait()
            dj = dst_rem[...]
            plsc.addupdate_scatter(deg_v, [dj], one16)
            pltpu.async_copy(rows_rem, agg_sh.at[dst_rem], sem_s[2], add=True)
            _scatter_wait(1)
            pltpu.make_async_copy(rows_rem, agg_sh.at[dst_rem],
                                  sem_s[2]).wait()
            plsc.subcore_barrier()

            for cq in range(2):
                q = cq * _NS + s

                @pl.when(q < _CQ)
                def _():
                    row = pl.multiple_of(q * _ZR, _ZR)
                    pltpu.async_copy(agg_sh.at[pl.ds(row, _ZR)],
                                     agg_out.at[r, c, pl.ds(row, _ZR)],
                                     sem_z)

            for q in range(_CQ):
                off = pl.multiple_of(((r * _CQ + q) * _NW + wid) * _ZR, 8)
                pltpu.async_copy(deg_v.at[pl.ds(q * _ZR, _ZR)],
                                 deg_out.at[pl.ds(off, _ZR)], sem_z)
            for cq in range(2):
                q = cq * _NS + s

                @pl.when(q < _CQ)
                def _():
                    row = pl.multiple_of(q * _ZR, _ZR)
                    pltpu.make_async_copy(agg_sh.at[pl.ds(row, _ZR)],
                                          agg_out.at[r, c, pl.ds(row, _ZR)],
                                          sem_z).wait()
            for q in range(_CQ):
                off = pl.multiple_of(((r * _CQ + q) * _NW + wid) * _ZR, 8)
                pltpu.make_async_copy(deg_v.at[pl.ds(q * _ZR, _ZR)],
                                      deg_out.at[pl.ds(off, _ZR)],
                                      sem_z).wait()

    return body(x, e0, e1, e2, zeros)


_BN = 400
_GB = _N // _BN


def _tc_dense(x, agg, deg, w_self, w_neigh, bias):
    def body(x_ref, agg_ref, deg_ref, ws_ref, wn_ref, b_ref, out_ref):
        xb = x_ref[...]
        ws = ws_ref[...]
        wn = wn_ref[...]
        b = b_ref[...]
        acc = jnp.dot(xb, ws[0] + ws[1] + ws[2],
                      preferred_element_type=jnp.float32)
        degs = jnp.sum(deg_ref[...], axis=(1, 2))
        inv = 1.0 / jnp.maximum(degs, 1.0)
        ag = agg_ref[...]
        for r in range(_R):
            h = (ag[r, 0] + ag[r, 1]) * inv[r][:, None]
            acc = acc + jnp.dot(h, wn[r], preferred_element_type=jnp.float32)
        acc = acc + (b[0] + b[1] + b[2])[None, :]
        out_ref[...] = acc

    return pl.pallas_call(
        body,
        grid=(_GB,),
        in_specs=[
            pl.BlockSpec((_BN, _D), lambda i: (i, 0)),
            pl.BlockSpec((_R, _NC, _BN, _D), lambda i: (0, 0, i, 0)),
            pl.BlockSpec((_R, 1, _NW, _ZR), lambda i: (0, i, 0, 0)),
            pl.BlockSpec((_R, _D, _D), lambda i: (0, 0, 0)),
            pl.BlockSpec((_R, _D, _D), lambda i: (0, 0, 0)),
            pl.BlockSpec((_R, _D), lambda i: (0, 0)),
        ],
        out_specs=pl.BlockSpec((_BN, _D), lambda i: (i, 0)),
        out_shape=jax.ShapeDtypeStruct((_N, _D), jnp.float32),
    )(x, agg, deg, w_self, w_neigh, bias)


def kernel(x, edge_index_r0, edge_index_r1, edge_index_r2,
           W_self_r0, W_neigh_r0, b_r0,
           W_self_r1, W_neigh_r1, b_r1,
           W_self_r2, W_neigh_r2, b_r2):
    e0 = edge_index_r0.reshape(-1)
    e1 = edge_index_r1.reshape(-1)
    e2 = edge_index_r2.reshape(-1)
    zeros = jnp.zeros((_ZR, _D), jnp.float32)
    agg, deg_flat = _sc_aggregate(x, e0, e1, e2, zeros)
    deg = deg_flat.reshape(_R, _CQ, _NW, _ZR)
    w_self = jnp.stack([W_self_r0, W_self_r1, W_self_r2])
    w_neigh = jnp.stack([W_neigh_r0, W_neigh_r1, W_neigh_r2])
    bias = jnp.stack([b_r0, b_r1, b_r2])
    return _tc_dense(x, agg, deg, w_self, w_neigh, bias)


# speedup vs baseline: 1.0556x; 1.0038x over previous
import functools

import jax
import jax.numpy as jnp
from jax import lax
from jax.experimental import pallas as pl
from jax.experimental.pallas import tpu as pltpu
from jax.experimental.pallas import tpu_sc as plsc

_N = 10000
_D = 128
_E = 320000
_R = 3
_K = 96
_NC = 2
_NS = 16
_NW = _NC * _NS
_EPT = _E // _NW
_FC = _EPT // _K
_REM = _EPT - _FC * _K
_NB = 3
_NG = 34
_ZR = 400
_CQ = _N // _ZR


def _sc_aggregate(x, e0, e1, e2, zeros):
    mesh = plsc.VectorSubcoreMesh(core_axis_name="c", subcore_axis_name="s")

    @functools.partial(
        pl.kernel,
        mesh=mesh,
        out_type=(
            jax.ShapeDtypeStruct((_R, _NC, _N, _D), jnp.float32),
            jax.ShapeDtypeStruct((_R * _CQ * _NW * _ZR,), jnp.float32),
        ),
        scratch_types=[
            [pltpu.VMEM((_K, _D), jnp.float32) for _ in range(_NB)],
            [pltpu.VMEM((_K,), jnp.int32) for _ in range(_NB)],
            [pltpu.VMEM((_K,), jnp.int32) for _ in range(_NB)],
            [pltpu.VMEM((_K,), jnp.int32) for _ in range(_NB)],
            pltpu.VMEM((_REM,), jnp.int32),
            pltpu.VMEM((_REM,), jnp.int32),
            pltpu.VMEM((_N,), jnp.float32),
            pltpu.VMEM_SHARED((_N, _D), jnp.float32),
            [pltpu.SemaphoreType.DMA for _ in range(_NB)],
            [pltpu.SemaphoreType.DMA for _ in range(_NB)],
            [pltpu.SemaphoreType.DMA for _ in range(_NB)],
            pltpu.SemaphoreType.DMA,
        ],
        compiler_params=pltpu.CompilerParams(needs_layout_passes=False),
    )
    def body(x_hbm, e0_hbm, e1_hbm, e2_hbm, zeros_hbm, agg_out, deg_out,
             rows, srcb, dstraw, dstb, src_rem, dst_rem, deg_v,
             agg_sh, sem_g, sem_s, sem_i, sem_z):
        c = lax.axis_index("c")
        s = lax.axis_index("s")
        wid = s * _NC + c
        z16 = jnp.zeros((16,), jnp.float32)
        one16 = jnp.ones((16,), jnp.float32)

        for r, e_hbm in enumerate((e0_hbm, e1_hbm, e2_hbm)):
            sbase = pl.multiple_of(wid * _EPT, 8)
            dbase = pl.multiple_of(_E + wid * _EPT, 8)

            def _idx_start(k, b):
                so = pl.multiple_of(sbase + k * _K, 8)
                do = pl.multiple_of(dbase + k * _K, 8)
                pltpu.async_copy(e_hbm.at[pl.ds(so, _K)], srcb[b], sem_i[b])
                pltpu.async_copy(e_hbm.at[pl.ds(do, _K)], dstraw[b], sem_i[b])

            def _idx_wait(k, b):
                so = pl.multiple_of(sbase + k * _K, 8)
                do = pl.multiple_of(dbase + k * _K, 8)
                pltpu.make_async_copy(e_hbm.at[pl.ds(so, _K)], srcb[b],
                                      sem_i[b]).wait()
                pltpu.make_async_copy(e_hbm.at[pl.ds(do, _K)], dstraw[b],
                                      sem_i[b]).wait()

            def _gather(b):
                pltpu.async_copy(x_hbm.at[srcb[b]], rows[b], sem_g[b])

            def _gather_wait(b):
                pltpu.make_async_copy(x_hbm.at[srcb[b]], rows[b],
                                      sem_g[b]).wait()

            def _scatter_wait(b):
                pltpu.make_async_copy(rows[b], agg_sh.at[dstb[b]],
                                      sem_s[b]).wait()

            def _stage(b):
                for j in range(_K // 16):
                    dj = dstraw[b][pl.ds(j * 16, 16)]
                    plsc.addupdate_scatter(deg_v, [dj], one16)
                    dstb[b][pl.ds(j * 16, 16)] = dj

            for b in range(_NB):
                _idx_start(b, b)
            _idx_wait(0, 0)
            _gather(0)

            for cq in range(2):
                q = cq * _NS + s

                @pl.when(q < _CQ)
                def _():
                    row = pl.multiple_of(q * _ZR, _ZR)
                    pltpu.async_copy(zeros_hbm, agg_sh.at[pl.ds(row, _ZR)],
                                     sem_z)

            def _zero_deg(i, carry):
                deg_v[pl.ds(i * 16, 16)] = z16
                return carry

            lax.fori_loop(0, _N // 16, _zero_deg, 0)
            for cq in range(2):
                q = cq * _NS + s

                @pl.when(q < _CQ)
                def _():
                    row = pl.multiple_of(q * _ZR, _ZR)
                    pltpu.make_async_copy(zeros_hbm,
                                          agg_sh.at[pl.ds(row, _ZR)],
                                          sem_z).wait()
            plsc.subcore_barrier()

            def _phase(k, b, g):
                nb = (b + 1) % _NB
                if b == 2:
                    _scatter_wait(nb)
                else:

                    @pl.when(g > 0)
                    def _():
                        _scatter_wait(nb)

                _idx_wait(k + 1, nb)
                _gather(nb)
                _gather_wait(b)
                _stage(b)
                pltpu.async_copy(rows[b], agg_sh.at[dstb[b]], sem_s[b],
                                 add=True)
                if b == 2:

                    @pl.when(g + 1 < _NG)
                    def _():
                        _idx_start(k + 3, b)
                else:
                    _idx_start(k + 3, b)

            def _group(g, carry):
                for b in range(_NB):
                    _phase(3 * g + b, b, g)
                return carry

            lax.fori_loop(0, _NG, _group, 0)

            ke = _NG * _NB
            _scatter_wait(1)
            _idx_wait(ke + 1, 1)
            _gather(1)
            _gather_wait(0)
            _stage(0)
            pltpu.async_copy(rows[0], agg_sh.at[dstb[0]], sem_s[0], add=True)
            _scatter_wait(2)
            _gather_wait(1)
            _stage(1)
            pltpu.async_copy(rows[1], agg_sh.at[dstb[1]], sem_s[1], add=True)
            _scatter_wait(0)
            roffs = pl.multiple_of(sbase + _FC * _K, 8)
            roffd = pl.multiple_of(dbase + _FC * _K, 8)
            pltpu.sync_copy(e_hbm.at[pl.ds(roffs, _REM)], src_rem)
            pltpu.sync_copy(e_hbm.at[pl.ds(roffd, _REM)], dst_rem)
            rows_rem = rows[2].at[pl.ds(0, _REM)]
            pltpu.async_copy(x_hbm.at[src_rem], rows_rem, sem_g[2]).wait()
            dj = dst_rem[...]
            plsc.addupdate_scatter(deg_v, [dj], one16)
            pltpu.async_copy(rows_rem, agg_sh.at[dst_rem], sem_s[2], add=True)
            _scatter_wait(1)
            pltpu.make_async_copy(rows_rem, agg_sh.at[dst_rem],
                                  sem_s[2]).wait()
            plsc.subcore_barrier()

            for cq in range(2):
                q = cq * _NS + s

                @pl.when(q < _CQ)
                def _():
                    row = pl.multiple_of(q * _ZR, _ZR)
                    pltpu.async_copy(agg_sh.at[pl.ds(row, _ZR)],
                                     agg_out.at[r, c, pl.ds(row, _ZR)],
                                     sem_z)

            for q in range(_CQ):
                off = pl.multiple_of(((r * _CQ + q) * _NW + wid) * _ZR, 8)
                pltpu.async_copy(deg_v.at[pl.ds(q * _ZR, _ZR)],
                                 deg_out.at[pl.ds(off, _ZR)], sem_z)
            for cq in range(2):
                q = cq * _NS + s

                @pl.when(q < _CQ)
                def _():
                    row = pl.multiple_of(q * _ZR, _ZR)
                    pltpu.make_async_copy(agg_sh.at[pl.ds(row, _ZR)],
                                          agg_out.at[r, c, pl.ds(row, _ZR)],
                                          sem_z).wait()
            for q in range(_CQ):
                off = pl.multiple_of(((r * _CQ + q) * _NW + wid) * _ZR, 8)
                pltpu.make_async_copy(deg_v.at[pl.ds(q * _ZR, _ZR)],
                                      deg_out.at[pl.ds(off, _ZR)],
                                      sem_z).wait()

    return body(x, e0, e1, e2, zeros)


_BN = 400
_GB = _N // _BN


def _tc_dense(x, agg, deg, ws0, ws1, ws2, wn0, wn1, wn2, b0, b1, b2):
    def body(x_ref, agg_ref, deg_ref, ws0_r, ws1_r, ws2_r,
             wn0_r, wn1_r, wn2_r, b0_r, b1_r, b2_r, out_ref):
        xb = x_ref[...]
        acc = jnp.dot(xb, ws0_r[...] + ws1_r[...] + ws2_r[...],
                      preferred_element_type=jnp.float32)
        degs = jnp.sum(deg_ref[...], axis=(1, 2))
        inv = 1.0 / jnp.maximum(degs, 1.0)
        ag = agg_ref[...]
        for r, wn_r in enumerate((wn0_r, wn1_r, wn2_r)):
            h = (ag[r, 0] + ag[r, 1]) * inv[r][:, None]
            acc = acc + jnp.dot(h, wn_r[...],
                                preferred_element_type=jnp.float32)
        acc = acc + (b0_r[...] + b1_r[...] + b2_r[...])[None, :]
        out_ref[...] = acc

    wspec = pl.BlockSpec((_D, _D), lambda i: (0, 0))
    bspec = pl.BlockSpec((_D,), lambda i: (0,))
    return pl.pallas_call(
        body,
        grid=(_GB,),
        in_specs=[
            pl.BlockSpec((_BN, _D), lambda i: (i, 0)),
            pl.BlockSpec((_R, _NC, _BN, _D), lambda i: (0, 0, i, 0)),
            pl.BlockSpec((_R, 1, _NW, _ZR), lambda i: (0, i, 0, 0)),
            wspec, wspec, wspec, wspec, wspec, wspec,
            bspec, bspec, bspec,
        ],
        out_specs=pl.BlockSpec((_BN, _D), lambda i: (i, 0)),
        out_shape=jax.ShapeDtypeStruct((_N, _D), jnp.float32),
    )(x, agg, deg, ws0, ws1, ws2, wn0, wn1, wn2, b0, b1, b2)


def kernel(x, edge_index_r0, edge_index_r1, edge_index_r2,
           W_self_r0, W_neigh_r0, b_r0,
           W_self_r1, W_neigh_r1, b_r1,
           W_self_r2, W_neigh_r2, b_r2):
    e0 = edge_index_r0.reshape(-1)
    e1 = edge_index_r1.reshape(-1)
    e2 = edge_index_r2.reshape(-1)
    zeros = jnp.zeros((_ZR, _D), jnp.float32)
    agg, deg_flat = _sc_aggregate(x, e0, e1, e2, zeros)
    deg = deg_flat.reshape(_R, _CQ, _NW, _ZR)
    return _tc_dense(x, agg, deg, W_self_r0, W_self_r1, W_self_r2,
                     W_neigh_r0, W_neigh_r1, W_neigh_r2, b_r0, b_r1, b_r2)
